# baseline (speedup 1.0000x reference)
import functools

import jax
import jax.numpy as jnp
from jax import lax
from jax.experimental import pallas as pl
from jax.experimental.pallas import tpu as pltpu
from jax.experimental.pallas import tpu_sc as plsc

BN = 80
SC_NODES = 4800
_NB = 3
_NTILES = 32
_C = 2
_L = 16


def _gat_block_tc(a1_ref, a2_ref, mb_a1_ref, mb_a2_ref, ft_ref, out_ref):
    a1 = a1_ref[...]
    a2 = a2_ref[...]
    mb_a1 = mb_a1_ref[...]
    mb_a2 = mb_a2_ref[...]

    b = a2[:, None, :] + mb_a1
    s = jnp.sum(a1[:, None, :] + mb_a2 + b, axis=-1, keepdims=True)
    z = s + b
    z = jnp.where(z >= 0, z, 0.01 * z)
    m = jnp.max(z, axis=1, keepdims=True)
    e = jnp.exp(z - m)
    denom = jnp.sum(e, axis=1)
    num = jnp.sum(e * ft_ref[...], axis=1)
    out_ref[...] = num / denom


def _tc_part(a1, a2, mb_a1, mb_a2, ft, n_tc):
    n, d = a1.shape
    k = mb_a1.shape[1]
    grid = (n_tc // BN,)
    return pl.pallas_call(
        _gat_block_tc,
        grid=grid,
        in_specs=[
            pl.BlockSpec((BN, d), lambda i: (i, 0)),
            pl.BlockSpec((BN, d), lambda i: (i, 0)),
            pl.BlockSpec((BN, k, d), lambda i: (i, 0, 0)),
            pl.BlockSpec((BN, k, d), lambda i: (i, 0, 0)),
            pl.BlockSpec((BN, k, d), lambda i: (i, 0, 0)),
        ],
        out_specs=pl.BlockSpec((BN, d), lambda i: (i, 0)),
        out_shape=jax.ShapeDtypeStruct((n_tc, d), jnp.float32),
    )(a1, a2, mb_a1, mb_a2, ft)


def _lane_sum(v, idxs):
    for idx in idxs:
        v = v + v.at[idx].get(mode="promise_in_bounds")
    return v


def _lane_max(v, idxs):
    for idx in idxs:
        v = jnp.maximum(v, v.at[idx].get(mode="promise_in_bounds"))
    return v


def _sc_body(n_tc, k_sz, d_sz,
             a1_hbm, a2_hbm, mb1_hbm, mb2_hbm, ft_hbm, out_hbm,
             a1_v, a2_v, mb1_v, mb2_v, ft_v, out_v,
             sem_in0, sem_in1, sem_in2, sem_out0, sem_out1, sem_out2):
    nchunk = d_sz // _L
    wid = lax.axis_index("s") * 2 + lax.axis_index("c")
    per_tile = SC_NODES // _NTILES
    nchunks = per_tile // _C
    ngroups = nchunks // _NB
    my_out0 = wid * per_tile
    my_in0 = n_tc + my_out0

    lanes = lax.iota(jnp.int32, _L)
    idxs = [jnp.bitwise_xor(lanes, sh) for sh in (8, 4, 2, 1)]

    in_sems = (sem_in0, sem_in1, sem_in2)
    out_sems = (sem_out0, sem_out1, sem_out2)

    def in_copies(b, i):
        ib = my_in0 + i * _C
        sem = in_sems[b]
        return [
            pltpu.make_async_copy(a1_hbm.at[pl.ds(ib, _C)], a1_v.at[b], sem),
            pltpu.make_async_copy(a2_hbm.at[pl.ds(ib, _C)], a2_v.at[b], sem),
            pltpu.make_async_copy(mb1_hbm.at[pl.ds(ib, _C)], mb1_v.at[b], sem),
            pltpu.make_async_copy(mb2_hbm.at[pl.ds(ib, _C)], mb2_v.at[b], sem),
            pltpu.make_async_copy(ft_hbm.at[pl.ds(ib, _C)], ft_v.at[b], sem),
        ]

    def start_in(b, i):
        for cp in in_copies(b, i):
            cp.start()

    def wait_in(b, i):
        for cp in in_copies(b, i):
            cp.wait()

    def out_copy(b, i):
        return pltpu.make_async_copy(
            out_v.at[b], out_hbm.at[pl.ds(my_out0 + i * _C, _C)], out_sems[b])

    def compute(b, i):
        for nl in range(_C):
            a2c = [a2_v[b, nl, pl.ds(c * _L, _L)] for c in range(nchunk)]
            s0 = a1_v[b, nl, pl.ds(0, _L)] + a2c[0]
            for c in range(1, nchunk):
                s0 = s0 + a1_v[b, nl, pl.ds(c * _L, _L)] + a2c[c]
            s0 = _lane_sum(s0, idxs)

            def k_body(kk, carry):
                m, den, acc = carry
                bs = []
                svec = jnp.zeros((_L,), jnp.float32)
                for c in range(nchunk):
                    mb1c = mb1_v[b, nl, kk, pl.ds(c * _L, _L)]
                    mb2c = mb2_v[b, nl, kk, pl.ds(c * _L, _L)]
                    bs.append(a2c[c] + mb1c)
                    svec = svec + mb1c + mb2c
                sk = _lane_sum(svec, idxs) + s0
                nm, nden, nacc = [], [], []
                for c in range(nchunk):
                    z = sk + bs[c]
                    z = jnp.maximum(z, 0.01 * z)
                    mi = jnp.maximum(m[c], z)
                    alpha = jnp.exp(m[c] - mi)
                    e = jnp.exp(z - mi)
                    nm.append(mi)
                    nden.append(den[c] * alpha + e)
                    ftc = ft_v[b, nl, kk, pl.ds(c * _L, _L)]
                    nacc.append(acc[c] * alpha + e * ftc)
                return tuple(nm), tuple(nden), tuple(nacc)

            minit = tuple(jnp.full((_L,), -jnp.inf, jnp.float32)
                          for _ in range(nchunk))
            zinit = tuple(jnp.zeros((_L,), jnp.float32) for _ in range(nchunk))
            m, den, acc = plsc.parallel_loop(
                0, k_sz, carry=(minit, zinit, zinit))(
                    lambda kk, carry: k_body(kk, carry))
            for c in range(nchunk):
                out_v[b, nl, pl.ds(c * _L, _L)] = acc[c] / den[c]

    start_in(0, 0)
    start_in(1, 1)

    def g_body(g, _):
        for b in range(_NB):
            i = _NB * g + b
            nxt = i + (_NB - 1)

            @pl.when(nxt < nchunks)
            def _():
                start_in((b + _NB - 1) % _NB, nxt)

            wait_in(b, i)

            @pl.when(g > 0)
            def _():
                out_copy(b, i - _NB).wait()

            compute(b, i)
            out_copy(b, i).start()
        return 0

    lax.fori_loop(0, ngroups, g_body, 0)
    for b in range(_NB):
        out_copy(b, nchunks - _NB + b).wait()


def _sc_part(a1, a2, mb_a1, mb_a2, ft, n_tc):
    n, d = a1.shape
    k = mb_a1.shape[1]
    mesh = plsc.VectorSubcoreMesh(core_axis_name="c", subcore_axis_name="s")
    f = functools.partial(
        pl.kernel,
        mesh=mesh,
        out_type=jax.ShapeDtypeStruct((SC_NODES, d), jnp.float32),
        scratch_types=[
            pltpu.VMEM((_NB, _C, d), jnp.float32),
            pltpu.VMEM((_NB, _C, d), jnp.float32),
            pltpu.VMEM((_NB, _C, k, d), jnp.float32),
            pltpu.VMEM((_NB, _C, k, d), jnp.float32),
            pltpu.VMEM((_NB, _C, k, d), jnp.float32),
            pltpu.VMEM((_NB, _C, d), jnp.float32),
            pltpu.SemaphoreType.DMA,
            pltpu.SemaphoreType.DMA,
            pltpu.SemaphoreType.DMA,
            pltpu.SemaphoreType.DMA,
            pltpu.SemaphoreType.DMA,
            pltpu.SemaphoreType.DMA,
        ],
    )(functools.partial(_sc_body, n_tc, k, d))
    return f(a1, a2, mb_a1, mb_a2, ft)


def kernel(a1, a2, mb_a1, mb_a2, ft):
    n = a1.shape[0]
    n_tc = n - SC_NODES
    out_tc = _tc_part(a1, a2, mb_a1, mb_a2, ft, n_tc)
    out_sc = _sc_part(a1, a2, mb_a1, mb_a2, ft, n_tc)
    return jnp.concatenate([out_tc, out_sc], axis=0)


# speedup vs baseline: 1.0683x; 1.0683x over previous
import functools

import jax
import jax.numpy as jnp
from jax import lax
from jax.experimental import pallas as pl
from jax.experimental.pallas import tpu as pltpu
from jax.experimental.pallas import tpu_sc as plsc

BN = 80
SC_NODES = 4480
_NTILES = 32
_C = 2
_L = 16


def _gat_block_tc(a1_ref, a2_ref, mb_a1_ref, mb_a2_ref, ft_ref, out_ref):
    a1 = a1_ref[...]
    a2 = a2_ref[...]
    mb_a1 = mb_a1_ref[...]
    mb_a2 = mb_a2_ref[...]

    b = a2[:, None, :] + mb_a1
    s = jnp.sum(a1[:, None, :] + mb_a2 + b, axis=-1, keepdims=True)
    z = s + b
    z = jnp.where(z >= 0, z, 0.01 * z)
    m = jnp.max(z, axis=1, keepdims=True)
    e = jnp.exp(z - m)
    denom = jnp.sum(e, axis=1)
    num = jnp.sum(e * ft_ref[...], axis=1)
    out_ref[...] = num / denom


def _tc_part(a1, a2, mb_a1, mb_a2, ft, n_tc):
    n, d = a1.shape
    k = mb_a1.shape[1]
    grid = (n_tc // BN,)
    return pl.pallas_call(
        _gat_block_tc,
        grid=grid,
        in_specs=[
            pl.BlockSpec((BN, d), lambda i: (i, 0)),
            pl.BlockSpec((BN, d), lambda i: (i, 0)),
            pl.BlockSpec((BN, k, d), lambda i: (i, 0, 0)),
            pl.BlockSpec((BN, k, d), lambda i: (i, 0, 0)),
            pl.BlockSpec((BN, k, d), lambda i: (i, 0, 0)),
        ],
        out_specs=pl.BlockSpec((BN, d), lambda i: (i, 0)),
        out_shape=jax.ShapeDtypeStruct((n_tc, d), jnp.float32),
    )(a1, a2, mb_a1, mb_a2, ft)


def _lane_sum(v, idxs):
    for idx in idxs:
        v = v + v.at[idx].get(mode="promise_in_bounds")
    return v


def _lane_max(v, idxs):
    for idx in idxs:
        v = jnp.maximum(v, v.at[idx].get(mode="promise_in_bounds"))
    return v


def _sc_body(n_tc, k_sz, d_sz,
             a1_hbm, a2_hbm, mb1_hbm, mb2_hbm, ft_hbm, out_hbm,
             a1_v, a2_v, mb1_v, mb2_v, ft_v, out_v,
             sem_in0, sem_in1, sem_out0, sem_out1):
    nchunk = d_sz // _L
    wid = lax.axis_index("s") * 2 + lax.axis_index("c")
    per_tile = SC_NODES // _NTILES
    nchunks = per_tile // _C
    halfn = nchunks // 2
    my_out0 = wid * per_tile
    my_in0 = n_tc + my_out0

    lanes = lax.iota(jnp.int32, _L)
    idxs = [jnp.bitwise_xor(lanes, sh) for sh in (8, 4, 2, 1)]

    in_sems = (sem_in0, sem_in1)
    out_sems = (sem_out0, sem_out1)

    def in_copies(b, i):
        ib = my_in0 + i * _C
        sem = in_sems[b]
        return [
            pltpu.make_async_copy(a1_hbm.at[pl.ds(ib, _C)], a1_v.at[b], sem),
            pltpu.make_async_copy(a2_hbm.at[pl.ds(ib, _C)], a2_v.at[b], sem),
            pltpu.make_async_copy(mb1_hbm.at[pl.ds(ib, _C)], mb1_v.at[b], sem),
            pltpu.make_async_copy(mb2_hbm.at[pl.ds(ib, _C)], mb2_v.at[b], sem),
            pltpu.make_async_copy(ft_hbm.at[pl.ds(ib, _C)], ft_v.at[b], sem),
        ]

    def start_in(b, i):
        for cp in in_copies(b, i):
            cp.start()

    def wait_in(b, i):
        for cp in in_copies(b, i):
            cp.wait()

    def out_copy(b, i):
        return pltpu.make_async_copy(
            out_v.at[b], out_hbm.at[pl.ds(my_out0 + i * _C, _C)], out_sems[b])

    def compute(b, i):
        for nl in range(_C):
            a2c = [a2_v[b, nl, pl.ds(c * _L, _L)] for c in range(nchunk)]
            s0 = a1_v[b, nl, pl.ds(0, _L)] + a2c[0]
            for c in range(1, nchunk):
                s0 = s0 + a1_v[b, nl, pl.ds(c * _L, _L)] + a2c[c]
            s0 = _lane_sum(s0, idxs)

            def k_body(kk, carry):
                m, den, acc = carry
                bs = []
                svec = jnp.zeros((_L,), jnp.float32)
                for c in range(nchunk):
                    mb1c = mb1_v[b, nl, kk, pl.ds(c * _L, _L)]
                    mb2c = mb2_v[b, nl, kk, pl.ds(c * _L, _L)]
                    bs.append(a2c[c] + mb1c)
                    svec = svec + mb1c + mb2c
                sk = _lane_sum(svec, idxs) + s0
                nm, nden, nacc = [], [], []
                for c in range(nchunk):
                    z = sk + bs[c]
                    z = jnp.maximum(z, 0.01 * z)
                    mi = jnp.maximum(m[c], z)
                    alpha = jnp.exp(m[c] - mi)
                    e = jnp.exp(z - mi)
                    nm.append(mi)
                    nden.append(den[c] * alpha + e)
                    ftc = ft_v[b, nl, kk, pl.ds(c * _L, _L)]
                    nacc.append(acc[c] * alpha + e * ftc)
                return tuple(nm), tuple(nden), tuple(nacc)

            minit = tuple(jnp.full((_L,), -jnp.inf, jnp.float32)
                          for _ in range(nchunk))
            zinit = tuple(jnp.zeros((_L,), jnp.float32) for _ in range(nchunk))
            m, den, acc = plsc.parallel_loop(
                0, k_sz, carry=(minit, zinit, zinit))(
                    lambda kk, carry: k_body(kk, carry))
            for c in range(nchunk):
                out_v[b, nl, pl.ds(c * _L, _L)] = acc[c] / den[c]

    start_in(0, 0)

    def g_body(g, _):
        i0 = 2 * g
        start_in(1, i0 + 1)
        wait_in(0, i0)

        @pl.when(g > 0)
        def _():
            out_copy(0, i0 - 2).wait()

        compute(0, i0)
        out_copy(0, i0).start()

        i1 = 2 * g + 1

        @pl.when(g < halfn - 1)
        def _():
            start_in(0, i1 + 1)

        wait_in(1, i1)

        @pl.when(g > 0)
        def _():
            out_copy(1, i1 - 2).wait()

        compute(1, i1)
        out_copy(1, i1).start()
        return 0

    lax.fori_loop(0, halfn, g_body, 0)
    out_copy(0, 2 * halfn - 2).wait()
    out_copy(1, 2 * halfn - 1).wait()


def _sc_part(a1, a2, mb_a1, mb_a2, ft, n_tc):
    n, d = a1.shape
    k = mb_a1.shape[1]
    mesh = plsc.VectorSubcoreMesh(core_axis_name="c", subcore_axis_name="s")
    f = functools.partial(
        pl.kernel,
        mesh=mesh,
        out_type=jax.ShapeDtypeStruct((SC_NODES, d), jnp.float32),
        scratch_types=[
            pltpu.VMEM((2, _C, d), jnp.float32),
            pltpu.VMEM((2, _C, d), jnp.float32),
            pltpu.VMEM((2, _C, k, d), jnp.float32),
            pltpu.VMEM((2, _C, k, d), jnp.float32),
            pltpu.VMEM((2, _C, k, d), jnp.float32),
            pltpu.VMEM((2, _C, d), jnp.float32),
            pltpu.SemaphoreType.DMA,
            pltpu.SemaphoreType.DMA,
            pltpu.SemaphoreType.DMA,
            pltpu.SemaphoreType.DMA,
        ],
    )(functools.partial(_sc_body, n_tc, k, d))
    return f(a1, a2, mb_a1, mb_a2, ft)


def kernel(a1, a2, mb_a1, mb_a2, ft):
    n = a1.shape[0]
    n_tc = n - SC_NODES
    out_tc = _tc_part(a1, a2, mb_a1, mb_a2, ft, n_tc)
    out_sc = _sc_part(a1, a2, mb_a1, mb_a2, ft, n_tc)
    return jnp.concatenate([out_tc, out_sc], axis=0)


# speedup vs baseline: 1.0807x; 1.0117x over previous
import functools

import jax
import jax.numpy as jnp
from jax import lax
from jax.experimental import pallas as pl
from jax.experimental.pallas import tpu as pltpu
from jax.experimental.pallas import tpu_sc as plsc

BN = 80
SC_NODES = 4480
_NTILES = 32
_C = 2
_L = 16


def _gat_block_tc(a1_ref, a2_ref, mb_a1_ref, mb_a2_ref, ft_ref, out_ref):
    a1 = a1_ref[...]
    a2 = a2_ref[...]
    mb_a1 = mb_a1_ref[...]
    mb_a2 = mb_a2_ref[...]

    b = a2[:, None, :] + mb_a1
    s = jnp.sum(a1[:, None, :] + mb_a2 + b, axis=-1, keepdims=True)
    z = s + b
    z = jnp.where(z >= 0, z, 0.01 * z)
    m = jnp.max(z, axis=1, keepdims=True)
    e = jnp.exp(z - m)
    denom = jnp.sum(e, axis=1)
    num = jnp.sum(e * ft_ref[...], axis=1)
    out_ref[...] = num / denom


def _tc_part(a1, a2, mb_a1, mb_a2, ft, n_tc):
    n, d = a1.shape
    k = mb_a1.shape[1]
    grid = (n_tc // BN,)
    return pl.pallas_call(
        _gat_block_tc,
        grid=grid,
        in_specs=[
            pl.BlockSpec((BN, d), lambda i: (i, 0)),
            pl.BlockSpec((BN, d), lambda i: (i, 0)),
            pl.BlockSpec((BN, k, d), lambda i: (i, 0, 0)),
            pl.BlockSpec((BN, k, d), lambda i: (i, 0, 0)),
            pl.BlockSpec((BN, k, d), lambda i: (i, 0, 0)),
        ],
        out_specs=pl.BlockSpec((BN, d), lambda i: (i, 0)),
        out_shape=jax.ShapeDtypeStruct((n, d), jnp.float32),
    )(a1, a2, mb_a1, mb_a2, ft)


def _lane_sum(v, idxs):
    for idx in idxs:
        v = v + v.at[idx].get(mode="promise_in_bounds")
    return v


def _lane_max(v, idxs):
    for idx in idxs:
        v = jnp.maximum(v, v.at[idx].get(mode="promise_in_bounds"))
    return v


def _sc_body(n_tc, k_sz, d_sz,
             a1_hbm, a2_hbm, mb1_hbm, mb2_hbm, ft_hbm, out_hbm,
             a1_v, a2_v, mb1_v, mb2_v, ft_v, out_v,
             sem_in0, sem_in1, sem_out0, sem_out1):
    nchunk = d_sz // _L
    wid = lax.axis_index("s") * 2 + lax.axis_index("c")
    per_tile = SC_NODES // _NTILES
    nchunks = per_tile // _C
    halfn = nchunks // 2
    my_out0 = wid * per_tile
    my_in0 = n_tc + my_out0

    lanes = lax.iota(jnp.int32, _L)
    idxs = [jnp.bitwise_xor(lanes, sh) for sh in (8, 4, 2, 1)]

    in_sems = (sem_in0, sem_in1)
    out_sems = (sem_out0, sem_out1)

    def in_copies(b, i):
        ib = my_in0 + i * _C
        sem = in_sems[b]
        return [
            pltpu.make_async_copy(a1_hbm.at[pl.ds(ib, _C)], a1_v.at[b], sem),
            pltpu.make_async_copy(a2_hbm.at[pl.ds(ib, _C)], a2_v.at[b], sem),
            pltpu.make_async_copy(mb1_hbm.at[pl.ds(ib, _C)], mb1_v.at[b], sem),
            pltpu.make_async_copy(mb2_hbm.at[pl.ds(ib, _C)], mb2_v.at[b], sem),
            pltpu.make_async_copy(ft_hbm.at[pl.ds(ib, _C)], ft_v.at[b], sem),
        ]

    def start_in(b, i):
        for cp in in_copies(b, i):
            cp.start()

    def wait_in(b, i):
        for cp in in_copies(b, i):
            cp.wait()

    def out_copy(b, i):
        return pltpu.make_async_copy(
            out_v.at[b], out_hbm.at[pl.ds(my_out0 + i * _C, _C)], out_sems[b])

    def compute(b, i):
        for nl in range(_C):
            a2c = [a2_v[b, nl, pl.ds(c * _L, _L)] for c in range(nchunk)]
            s0 = a1_v[b, nl, pl.ds(0, _L)] + a2c[0]
            for c in range(1, nchunk):
                s0 = s0 + a1_v[b, nl, pl.ds(c * _L, _L)] + a2c[c]
            s0 = _lane_sum(s0, idxs)

            def k_body(kk, carry):
                m, den, acc = carry
                bs = []
                svec = jnp.zeros((_L,), jnp.float32)
                for c in range(nchunk):
                    mb1c = mb1_v[b, nl, kk, pl.ds(c * _L, _L)]
                    mb2c = mb2_v[b, nl, kk, pl.ds(c * _L, _L)]
                    bs.append(a2c[c] + mb1c)
                    svec = svec + mb1c + mb2c
                sk = _lane_sum(svec, idxs) + s0
                nm, nden, nacc = [], [], []
                for c in range(nchunk):
                    z = sk + bs[c]
                    z = jnp.maximum(z, 0.01 * z)
                    mi = jnp.maximum(m[c], z)
                    alpha = jnp.exp(m[c] - mi)
                    e = jnp.exp(z - mi)
                    nm.append(mi)
                    nden.append(den[c] * alpha + e)
                    ftc = ft_v[b, nl, kk, pl.ds(c * _L, _L)]
                    nacc.append(acc[c] * alpha + e * ftc)
                return tuple(nm), tuple(nden), tuple(nacc)

            minit = tuple(jnp.full((_L,), -jnp.inf, jnp.float32)
                          for _ in range(nchunk))
            zinit = tuple(jnp.zeros((_L,), jnp.float32) for _ in range(nchunk))
            m, den, acc = plsc.parallel_loop(
                0, k_sz, carry=(minit, zinit, zinit))(
                    lambda kk, carry: k_body(kk, carry))
            for c in range(nchunk):
                out_v[b, nl, pl.ds(c * _L, _L)] = acc[c] / den[c]

    start_in(0, 0)

    def g_body(g, _):
        i0 = 2 * g
        start_in(1, i0 + 1)
        wait_in(0, i0)

        @pl.when(g > 0)
        def _():
            out_copy(0, i0 - 2).wait()

        compute(0, i0)
        out_copy(0, i0).start()

        i1 = 2 * g + 1

        @pl.when(g < halfn - 1)
        def _():
            start_in(0, i1 + 1)

        wait_in(1, i1)

        @pl.when(g > 0)
        def _():
            out_copy(1, i1 - 2).wait()

        compute(1, i1)
        out_copy(1, i1).start()
        return 0

    lax.fori_loop(0, halfn, g_body, 0)
    out_copy(0, 2 * halfn - 2).wait()
    out_copy(1, 2 * halfn - 1).wait()


def _sc_part(a1, a2, mb_a1, mb_a2, ft, n_tc):
    n, d = a1.shape
    k = mb_a1.shape[1]
    mesh = plsc.VectorSubcoreMesh(core_axis_name="c", subcore_axis_name="s")
    f = functools.partial(
        pl.kernel,
        mesh=mesh,
        out_type=jax.ShapeDtypeStruct((SC_NODES, d), jnp.float32),
        scratch_types=[
            pltpu.VMEM((2, _C, d), jnp.float32),
            pltpu.VMEM((2, _C, d), jnp.float32),
            pltpu.VMEM((2, _C, k, d), jnp.float32),
            pltpu.VMEM((2, _C, k, d), jnp.float32),
            pltpu.VMEM((2, _C, k, d), jnp.float32),
            pltpu.VMEM((2, _C, d), jnp.float32),
            pltpu.SemaphoreType.DMA,
            pltpu.SemaphoreType.DMA,
            pltpu.SemaphoreType.DMA,
            pltpu.SemaphoreType.DMA,
        ],
    )(functools.partial(_sc_body, n_tc, k, d))
    return f(a1, a2, mb_a1, mb_a2, ft)


def kernel(a1, a2, mb_a1, mb_a2, ft):
    n = a1.shape[0]
    n_tc = n - SC_NODES
    out_tc = _tc_part(a1, a2, mb_a1, mb_a2, ft, n_tc)
    out_sc = _sc_part(a1, a2, mb_a1, mb_a2, ft, n_tc)
    return lax.dynamic_update_slice(out_tc, out_sc, (n_tc, 0))
